# baseline (speedup 1.0000x reference)
import jax
import jax.numpy as jnp
from jax import lax
from jax.experimental import pallas as pl
from jax.experimental.pallas import tpu as pltpu
from jax.experimental.pallas import tpu_sc as plsc

S = 64
POP = 100000
D = 128
L = 16

NCORES = 2
NSUB = 16
RG = 8
NGRP = S // RG
NCHK = 4
QCOLS = 4992
TAILC = 160
C0 = POP - NCHK * QCOLS - TAILC
TAIL0 = POP - TAILC
TITER = QCOLS // 128

TCB = 1664
NTCB = C0 // TCB
INT_BIG = 2**31 - 1


def _sc_body(f_hbm, scv_hbm, sci_hbm, buf, tailbuf, stage, idxbuf,
             mvals, midx, fvm, ivm, svals, sidx, sem0, semt):
    c = lax.axis_index("c")
    sid = lax.axis_index("s")
    grp = c * (NGRP // NCORES) + sid // NCHK
    q = sid % NCHK
    row0 = pl.multiple_of(grp * RG, 8)

    lane = lax.broadcasted_iota(jnp.int32, (L,), 0)

    tail_copy = pltpu.async_copy(
        f_hbm.at[pl.ds(row0, RG), pl.ds(TAIL0, TAILC)], tailbuf, semt)
    cb = pl.multiple_of(C0 + q * QCOLS, 128)
    main_copy = pltpu.async_copy(
        f_hbm.at[pl.ds(row0, RG), pl.ds(cb, QCOLS)], buf, sem0)

    neg_inf = jnp.broadcast_to(jnp.float32(-jnp.inf), (L,))
    zero = jnp.broadcast_to(jnp.int32(0), (L,))
    pb0 = C0 // L + q * (QCOLS // L)

    main_copy.wait()

    def body(t, carry):
        ms = list(carry[:RG])
        xs = list(carry[RG:])
        base = pb0 + t * 8
        for i in range(8):
            g = jnp.broadcast_to(jnp.int32(0) + (base + i), (L,))
            for r in range(RG):
                v = buf[r, pl.ds(t * 128 + i * L, L)]
                cond = v > ms[r]
                ms[r] = jnp.where(cond, v, ms[r])
                xs[r] = jnp.where(cond, g, xs[r])
        return (*ms, *xs)

    out = lax.fori_loop(0, TITER, body, (*([neg_inf] * RG), *([zero] * RG)))
    ms = list(out[:RG])
    xs = list(out[RG:])

    tail_copy.wait()
    for i in range(TAILC // L):
        g = jnp.broadcast_to(jnp.int32(TAIL0 // L + i), (L,))
        for r in range(RG):
            v = tailbuf[r, pl.ds(i * L, L)]
            cond = v > ms[r]
            ms[r] = jnp.where(cond, v, ms[r])
            xs[r] = jnp.where(cond, g, xs[r])

    valvec = jnp.broadcast_to(jnp.float32(0.0), (L,))
    idxvec = zero
    for r in range(RG):
        mv = ms[r]
        gv = xs[r] * L + lane
        for sh in (8, 4, 2, 1):
            perm = (lane + sh) & (L - 1)
            v2 = mv.at[perm].get(mode="promise_in_bounds")
            x2 = gv.at[perm].get(mode="promise_in_bounds")
            better = (v2 > mv) | ((v2 == mv) & (x2 < gv))
            mv = jnp.where(better, v2, mv)
            gv = jnp.where(better, x2, gv)
        valvec = jnp.where(lane == r, mv, valvec)
        idxvec = jnp.where(lane == r, gv, idxvec)

    stage[...] = valvec
    idxbuf[...] = idxvec
    pltpu.sync_copy(stage, svals.at[pl.ds(pl.multiple_of(sid * L, 8), L)])
    pltpu.sync_copy(idxbuf, sidx.at[pl.ds(pl.multiple_of(sid * L, 8), L)])
    plsc.subcore_barrier()

    @pl.when(q == 0)
    def _merge_and_emit():
        base = pl.multiple_of((sid - q) * L, 8)
        pltpu.sync_copy(svals.at[pl.ds(base, NCHK * L)], mvals)
        pltpu.sync_copy(sidx.at[pl.ds(base, NCHK * L)], midx)
        mv = mvals[pl.ds(0, L)]
        gv = midx[pl.ds(0, L)]
        for qq in range(1, NCHK):
            v2 = mvals[pl.ds(qq * L, L)]
            x2 = midx[pl.ds(qq * L, L)]
            better = (v2 > mv) | ((v2 == mv) & (x2 < gv))
            mv = jnp.where(better, v2, mv)
            gv = jnp.where(better, x2, gv)
        for r in range(RG):
            perm = (lane + r) & (L - 1)
            fvm[r, pl.ds(0, L)] = mv.at[perm].get(mode="promise_in_bounds")
            ivm[r, pl.ds(0, L)] = gv.at[perm].get(mode="promise_in_bounds")
        pltpu.sync_copy(fvm, scv_hbm.at[pl.ds(row0, RG)])
        pltpu.sync_copy(ivm, sci_hbm.at[pl.ds(row0, RG)])


def _sc_scan(f_samples):
    mesh = plsc.VectorSubcoreMesh(core_axis_name="c", subcore_axis_name="s")
    kfn = pl.kernel(
        _sc_body,
        out_type=[
            jax.ShapeDtypeStruct((S, D), jnp.float32),
            jax.ShapeDtypeStruct((S, D), jnp.int32),
        ],
        mesh=mesh,
        scratch_types=[
            pltpu.VMEM((RG, QCOLS), jnp.float32),
            pltpu.VMEM((RG, TAILC), jnp.float32),
            pltpu.VMEM((L,), jnp.float32),
            pltpu.VMEM((L,), jnp.int32),
            pltpu.VMEM((NCHK * L,), jnp.float32),
            pltpu.VMEM((NCHK * L,), jnp.int32),
            pltpu.VMEM((RG, D), jnp.float32),
            pltpu.VMEM((RG, D), jnp.int32),
            pltpu.VMEM_SHARED((NSUB * L,), jnp.float32),
            pltpu.VMEM_SHARED((NSUB * L,), jnp.int32),
            pltpu.SemaphoreType.DMA,
            pltpu.SemaphoreType.DMA,
        ],
    )
    return kfn(f_samples)


def _tc_scan_body(f_ref, vout_ref, iout_ref, macc, iacc):
    i = pl.program_id(0)

    @pl.when(i == 0)
    def _init():
        macc[...] = jnp.full((S, 128), -jnp.inf, jnp.float32)
        iacc[...] = jnp.zeros((S, 128), jnp.int32)

    blk = f_ref[...]
    m = macc[...]
    x = iacc[...]
    colbase = lax.broadcasted_iota(jnp.int32, (S, 128), 1) + i * TCB
    for j in range(TCB // 128):
        v = blk[:, j * 128:(j + 1) * 128]
        col = colbase + j * 128
        cond = v > m
        m = jnp.where(cond, v, m)
        x = jnp.where(cond, col, x)
    macc[...] = m
    iacc[...] = x

    @pl.when(i == NTCB - 1)
    def _fin():
        rowmax = jnp.max(m, axis=1, keepdims=True)
        cand = jnp.where(m == rowmax, x, jnp.int32(INT_BIG))
        vout_ref[...] = rowmax
        iout_ref[...] = jnp.min(cand, axis=1, keepdims=True)


def _tc_scan(f_samples):
    return pl.pallas_call(
        _tc_scan_body,
        grid=(NTCB,),
        in_specs=[pl.BlockSpec((S, TCB), lambda i: (0, i))],
        out_specs=[pl.BlockSpec((S, 1), lambda i: (0, 0)),
                   pl.BlockSpec((S, 1), lambda i: (0, 0))],
        out_shape=[jax.ShapeDtypeStruct((S, 1), jnp.float32),
                   jax.ShapeDtypeStruct((S, 1), jnp.int32)],
        scratch_shapes=[pltpu.VMEM((S, 128), jnp.float32),
                        pltpu.VMEM((S, 128), jnp.int32)],
    )(f_samples)


def _tc_merge_body(scv_ref, sci_ref, tcv_ref, tci_ref, fmax_ref, fidx_ref):
    scv = scv_ref[:, 0:1]
    sci = sci_ref[:, 0:1]
    tcv = tcv_ref[...]
    tci = tci_ref[...]
    better = scv > tcv
    fmax_ref[...] = jnp.where(better, scv, tcv)
    fidx_ref[...] = jnp.where(better, sci, tci)


def _tc_merge(scv, sci, tcv, tci):
    return pl.pallas_call(
        _tc_merge_body,
        out_shape=[jax.ShapeDtypeStruct((S, 1), jnp.float32),
                   jax.ShapeDtypeStruct((S, 1), jnp.int32)],
    )(scv, sci, tcv, tci)


def _tc_gather_body(fidx_ref, x_ref, xout_ref, sem):
    copies = []
    for i in range(S):
        idx = fidx_ref[i, 0]
        cp = pltpu.make_async_copy(x_ref.at[idx], xout_ref.at[i],
                                   sem.at[i % 8])
        cp.start()
        copies.append(cp)
    for cp in copies:
        cp.wait()


def _tc_gather(fidx, x_cand):
    return pl.pallas_call(
        _tc_gather_body,
        in_specs=[pl.BlockSpec(memory_space=pltpu.SMEM),
                  pl.BlockSpec(memory_space=pl.ANY)],
        out_shape=jax.ShapeDtypeStruct((S, D), jnp.float32),
        scratch_shapes=[pltpu.SemaphoreType.DMA((8,))],
    )(fidx, x_cand)


@jax.jit
def _run(x_cand, f_samples):
    scv, sci = _sc_scan(f_samples)
    tcv, tci = _tc_scan(f_samples)
    fmax, fidx = _tc_merge(scv, sci, tcv, tci)
    x_max = _tc_gather(fidx, x_cand)
    return x_max, fmax.reshape(S)


def kernel(X_cand, f_samples, num_samples):
    return _run(X_cand, f_samples)
